# Initial kernel scaffold; baseline (speedup 1.0000x reference)
import jax
import jax.numpy as jnp
from jax.experimental import pallas as pl


def kernel(x, thresholds, y):
    raise NotImplementedError("write your pallas kernel here")


# speedup vs baseline: 6051.0720x; 6051.0720x over previous
import functools

import jax
import jax.numpy as jnp
from jax.experimental import pallas as pl
from jax.experimental.pallas import tpu as pltpu

_K = 32


def _codec_block_kernel(x_ref, t_ref, y_ref, o_ref):
    xb = x_ref[...]
    t = t_ref[...]
    yv = y_ref[...]

    nodes = [yv[k : k + 1, :] for k in range(_K)]
    size = 1
    while len(nodes) > 1:
        nxt = []
        for j in range(len(nodes) // 2):
            m = 2 * size * j + size - 1
            mask = xb > t[m : m + 1, :]
            nxt.append(jnp.where(mask, nodes[2 * j + 1], nodes[2 * j]))
        nodes = nxt
        size *= 2
    o_ref[...] = nodes[0]


@functools.partial(jax.jit, static_argnames=("tn",))
def _codec(x2, t_pad, y_t, tn):
    n, h = x2.shape
    grid = (n // tn,)
    return pl.pallas_call(
        _codec_block_kernel,
        grid=grid,
        in_specs=[
            pl.BlockSpec((tn, h), lambda i: (i, 0)),
            pl.BlockSpec((_K, h), lambda i: (0, 0)),
            pl.BlockSpec((_K, h), lambda i: (0, 0)),
        ],
        out_specs=pl.BlockSpec((tn, h), lambda i: (i, 0)),
        out_shape=jax.ShapeDtypeStruct((n, h), jnp.float32),
        compiler_params=pltpu.CompilerParams(
            dimension_semantics=("arbitrary",),
        ),
    )(x2, t_pad, y_t)


def kernel(x, thresholds, y):
    shape = x.shape
    h = shape[-1]
    x2 = x.reshape(-1, h).astype(jnp.float32)
    t_t = thresholds.T.astype(jnp.float32)
    t_pad = jnp.concatenate([t_t, t_t[-1:, :]], axis=0)
    y_t = y.T.astype(jnp.float32)
    out = _codec(x2, t_pad, y_t, tn=256)
    return out.reshape(shape)


# speedup vs baseline: 8005.2491x; 1.3229x over previous
import functools

import jax
import jax.numpy as jnp
from jax.experimental import pallas as pl
from jax.experimental.pallas import tpu as pltpu

_K = 32


def _subtree(xb, t, yv, k0, k1):
    nodes = [yv[k : k + 1, :] for k in range(k0, k1)]
    size = 1
    while len(nodes) > 1:
        nxt = []
        for j in range(len(nodes) // 2):
            m = k0 + 2 * size * j + size - 1
            mask = xb > t[m : m + 1, :]
            nxt.append(jnp.where(mask, nodes[2 * j + 1], nodes[2 * j]))
        nodes = nxt
        size *= 2
    return nodes[0]


def _codec_block_kernel(x_ref, t_ref, y_ref, o_ref):
    xb = x_ref[...]
    t = t_ref[...]
    yv = y_ref[...]
    lo = _subtree(xb, t, yv, 0, _K // 2)
    hi = _subtree(xb, t, yv, _K // 2, _K)
    o_ref[...] = jnp.where(xb > t[_K // 2 - 1 : _K // 2, :], hi, lo)


@functools.partial(jax.jit, static_argnames=("tn", "th"))
def _codec(x2, t_pad, y_t, tn, th):
    n, h = x2.shape
    grid = (h // th, n // tn)
    return pl.pallas_call(
        _codec_block_kernel,
        grid=grid,
        in_specs=[
            pl.BlockSpec((tn, th), lambda j, i: (i, j)),
            pl.BlockSpec((_K, th), lambda j, i: (0, j)),
            pl.BlockSpec((_K, th), lambda j, i: (0, j)),
        ],
        out_specs=pl.BlockSpec((tn, th), lambda j, i: (i, j)),
        out_shape=jax.ShapeDtypeStruct((n, h), jnp.float32),
        compiler_params=pltpu.CompilerParams(
            dimension_semantics=("arbitrary", "arbitrary"),
        ),
    )(x2, t_pad, y_t)


def kernel(x, thresholds, y):
    shape = x.shape
    h = shape[-1]
    x2 = x.reshape(-1, h).astype(jnp.float32)
    t_t = thresholds.T.astype(jnp.float32)
    t_pad = jnp.concatenate([t_t, t_t[-1:, :]], axis=0)
    y_t = y.T.astype(jnp.float32)
    out = _codec(x2, t_pad, y_t, tn=2048, th=128)
    return out.reshape(shape)
